# Initial kernel scaffold; baseline (speedup 1.0000x reference)
import jax
import jax.numpy as jnp
from jax.experimental import pallas as pl


def kernel(x, edge_index, edge_attr, batch, conv_Wl, conv_bl, conv_Wr, Wc1, bc1, Wcls, bcls, Wf, bf):
    raise NotImplementedError("write your pallas kernel here")


# speedup vs baseline: 3.9333x; 3.9333x over previous
import functools

import jax
import jax.numpy as jnp
from jax import lax
from jax.experimental import pallas as pl
from jax.experimental.pallas import tpu as pltpu
from jax.experimental.pallas import tpu_sc as plsc

N = 10000
E = 320000
F = 128
MAXDEG = 10
NG = 500
NGP = 512

NC = 2
NS = 16
NW = NC * NS
EPW = E // NW
CHUNK = 80
NCHUNK = EPW // CHUNK
NPAD = 10240
RPS = NPAD // NS


def _sc_segsum_body(h_hbm, src_hbm, dst_hbm, zeros_hbm, out_hbm,
                    src_v, dst_v, rows_v, agg_sh, sem):
    c = lax.axis_index("c")
    s = lax.axis_index("s")
    pltpu.sync_copy(zeros_hbm.at[pl.ds(s * RPS, RPS)],
                    agg_sh.at[pl.ds(s * RPS, RPS)])
    plsc.subcore_barrier()

    wid = s * NC + c
    base = wid * EPW

    def body(i, carry):
        off = base + i * CHUNK
        pltpu.sync_copy(src_hbm.at[pl.ds(off, CHUNK)], src_v)
        pltpu.sync_copy(dst_hbm.at[pl.ds(off, CHUNK)], dst_v)
        pltpu.async_copy(h_hbm.at[src_v], rows_v, sem).wait()
        pltpu.sync_copy(rows_v, agg_sh.at[dst_v], add=True)
        return carry

    lax.fori_loop(0, NCHUNK, body, 0)
    plsc.subcore_barrier()
    pltpu.sync_copy(agg_sh.at[pl.ds(s * RPS, RPS)],
                    out_hbm.at[c, pl.ds(s * RPS, RPS)])


def _sc_degcount_body(dst_hbm, zeros_hbm, ones_hbm, out_hbm, dst_v, ones_v,
                      deg_sh):
    c = lax.axis_index("c")
    s = lax.axis_index("s")
    pltpu.sync_copy(zeros_hbm.at[pl.ds(s * RPS, RPS)],
                    deg_sh.at[pl.ds(s * RPS, RPS)])
    pltpu.sync_copy(ones_hbm, ones_v)
    plsc.subcore_barrier()

    wid = s * NC + c
    base = wid * EPW

    def body(i, carry):
        off = base + i * CHUNK
        pltpu.sync_copy(dst_hbm.at[pl.ds(off, CHUNK)], dst_v)
        pltpu.sync_copy(ones_v, deg_sh.at[dst_v], add=True)
        return carry

    lax.fori_loop(0, NCHUNK, body, 0)
    plsc.subcore_barrier()
    pltpu.sync_copy(deg_sh.at[pl.ds(s * RPS, RPS)],
                    out_hbm.at[c, pl.ds(s * RPS, RPS)])


@functools.lru_cache(maxsize=None)
def _sc_kernels():
    mesh = plsc.VectorSubcoreMesh(core_axis_name="c", subcore_axis_name="s",
                                  num_cores=NC, num_subcores=NS)
    segsum = pl.kernel(
        _sc_segsum_body,
        out_type=jax.ShapeDtypeStruct((NC, NPAD, F), jnp.float32),
        mesh=mesh,
        scratch_types=[
            pltpu.VMEM((CHUNK,), jnp.int32),
            pltpu.VMEM((CHUNK,), jnp.int32),
            pltpu.VMEM((CHUNK, F), jnp.float32),
            pltpu.VMEM_SHARED((NPAD, F), jnp.float32),
            pltpu.SemaphoreType.DMA,
        ],
    )
    degcount = pl.kernel(
        _sc_degcount_body,
        out_type=jax.ShapeDtypeStruct((NC, NPAD, F), jnp.float32),
        mesh=mesh,
        scratch_types=[
            pltpu.VMEM((CHUNK,), jnp.int32),
            pltpu.VMEM((CHUNK, F), jnp.float32),
            pltpu.VMEM_SHARED((NPAD, F), jnp.float32),
        ],
    )
    return segsum, degcount


def _sc_segsum(h, src, dst, zeros):
    return _sc_kernels()[0](h, src, dst, zeros)


def _sc_degcount(dst, zeros):
    ones = jnp.ones((CHUNK, F), jnp.float32)
    return _sc_kernels()[1](dst, zeros, ones)[:, :, :16]


BLK = 1000
NBLK = N // BLK


def _leaky(v):
    return jnp.where(v >= 0, v, 0.01 * v)


def _tc_layer_body(agg_ref, h_ref, degp_ref, Wl_ref, bl_ref, Wr_ref, out_ref):
    agg = agg_ref[0] + agg_ref[1]
    deg = degp_ref[0, :, 0:1] + degp_ref[1, :, 0:1]
    deg = jnp.clip(deg, 0.0, float(MAXDEG))
    h = h_ref[...]
    acc = jnp.zeros((BLK, F), jnp.float32)
    for d in range(MAXDEG + 1):
        r = (jnp.dot(agg, Wl_ref[d], preferred_element_type=jnp.float32)
             + jnp.dot(h, Wr_ref[d], preferred_element_type=jnp.float32)
             + bl_ref[d][None, :])
        acc = jnp.where(deg == float(d), r, acc)
    out_ref[...] = _leaky(acc)


_tc_layer = pl.pallas_call(
    _tc_layer_body,
    grid=(NBLK,),
    in_specs=[
        pl.BlockSpec((NC, BLK, F), lambda i: (0, i, 0)),
        pl.BlockSpec((BLK, F), lambda i: (i, 0)),
        pl.BlockSpec((NC, BLK, 16), lambda i: (0, i, 0)),
        pl.BlockSpec((MAXDEG + 1, F, F), lambda i: (0, 0, 0)),
        pl.BlockSpec((MAXDEG + 1, F), lambda i: (0, 0)),
        pl.BlockSpec((MAXDEG + 1, F, F), lambda i: (0, 0, 0)),
    ],
    out_specs=pl.BlockSpec((BLK, F), lambda i: (i, 0)),
    out_shape=jax.ShapeDtypeStruct((N, F), jnp.float32),
)


def _tc_pool_body(h_ref, batch_ref, out_ref):
    @pl.when(pl.program_id(0) == 0)
    def _():
        out_ref[...] = jnp.zeros_like(out_ref)

    gids = lax.broadcasted_iota(jnp.int32, (1, NGP), 1)
    oh = (batch_ref[...] == gids).astype(jnp.float32)
    out_ref[...] += lax.dot_general(
        oh, h_ref[...], (((0,), (0,)), ((), ())),
        preferred_element_type=jnp.float32)


_tc_pool = pl.pallas_call(
    _tc_pool_body,
    grid=(NBLK,),
    in_specs=[
        pl.BlockSpec((BLK, F), lambda i: (i, 0)),
        pl.BlockSpec((BLK, 1), lambda i: (i, 0)),
    ],
    out_specs=pl.BlockSpec((NGP, F), lambda i: (0, 0)),
    out_shape=jax.ShapeDtypeStruct((NGP, F), jnp.float32),
)


def _tc_mlp_body(h1_ref, h2_ref, h3_ref, batch_ref, pool_ref,
                 Wc1_ref, bc1_ref, Wcls_ref, bcls_ref, wf_ref, bf_ref,
                 out_ref):
    gids = lax.broadcasted_iota(jnp.int32, (1, NGP), 1)
    oh = (batch_ref[...] == gids).astype(jnp.float32)
    hp = jnp.dot(oh, pool_ref[...], preferred_element_type=jnp.float32)
    z = (jnp.dot(h1_ref[...], Wc1_ref[0:F], preferred_element_type=jnp.float32)
         + jnp.dot(h2_ref[...], Wc1_ref[F:2 * F], preferred_element_type=jnp.float32)
         + jnp.dot(h3_ref[...], Wc1_ref[2 * F:3 * F], preferred_element_type=jnp.float32)
         + jnp.dot(hp, Wc1_ref[3 * F:4 * F], preferred_element_type=jnp.float32)
         + bc1_ref[0][None, :])
    for i in range(2):
        z = _leaky(jnp.dot(z, Wcls_ref[i], preferred_element_type=jnp.float32)
                   + bcls_ref[i][None, :])
    o = jnp.sum(z * wf_ref[...], axis=1, keepdims=True) + bf_ref[0, 0]
    o = 1.0 / (1.0 + jnp.exp(-o))
    out_ref[...] = jnp.broadcast_to(o, (BLK, F))


def _make_mlp():
    return pl.pallas_call(
        _tc_mlp_body,
        grid=(NBLK,),
        in_specs=[
            pl.BlockSpec((BLK, F), lambda i: (i, 0)),
            pl.BlockSpec((BLK, F), lambda i: (i, 0)),
            pl.BlockSpec((BLK, F), lambda i: (i, 0)),
            pl.BlockSpec((BLK, 1), lambda i: (i, 0)),
            pl.BlockSpec((NGP, F), lambda i: (0, 0)),
            pl.BlockSpec((4 * F, F), lambda i: (0, 0)),
            pl.BlockSpec((1, F), lambda i: (0, 0)),
            pl.BlockSpec((2, F, F), lambda i: (0, 0, 0)),
            pl.BlockSpec((2, F), lambda i: (0, 0)),
            pl.BlockSpec((1, F), lambda i: (0, 0)),
            pl.BlockSpec((1, 1), lambda i: (0, 0)),
        ],
        out_specs=pl.BlockSpec((BLK, F), lambda i: (i, 0)),
        out_shape=jax.ShapeDtypeStruct((N, F), jnp.float32),
    )


_tc_mlp = _make_mlp()


def kernel(x, edge_index, edge_attr, batch, conv_Wl, conv_bl, conv_Wr,
           Wc1, bc1, Wcls, bcls, Wf, bf):
    src = edge_index[0]
    dst = edge_index[1]
    zeros_f = jnp.zeros((NPAD, F), jnp.float32)

    degp = _sc_degcount(dst, zeros_f)
    batch2 = batch.reshape(N, 1)

    hs = []
    h = x
    for layer in range(3):
        aggp = _sc_segsum(h, src, dst, zeros_f)
        h = _tc_layer(aggp, h, degp, conv_Wl[layer], conv_bl[layer],
                      conv_Wr[layer])
        hs.append(h)

    pool = _tc_pool(hs[2], batch2)

    out = _tc_mlp(hs[0], hs[1], hs[2], batch2, pool, Wc1,
                  bc1.reshape(1, F), Wcls, bcls.reshape(2, F),
                  Wf.reshape(1, F), bf.reshape(1, 1))
    return out[:, :1]


# speedup vs baseline: 7.9398x; 2.0186x over previous
import functools

import jax
import jax.numpy as jnp
from jax import lax
from jax.experimental import pallas as pl
from jax.experimental.pallas import tpu as pltpu
from jax.experimental.pallas import tpu_sc as plsc

N = 10000
E = 320000
F = 128
MAXDEG = 10
NG = 500
NGP = 512

NC = 2
NS = 16
NW = NC * NS
EPW = E // NW
CHUNK = 80
NCHUNK = EPW // CHUNK
NBUF = 4
NFULL = NCHUNK // NBUF
NPAD = 10240
RPS = NPAD // NS


def _sc_segsum_body(h_hbm, src_hbm, dst_hbm, zeros_hbm, out_hbm,
                    sb0, sb1, sb2, sb3, db0, db1, db2, db3,
                    r0, r1, r2, r3, agg_sh, isem, gsem):
    c = lax.axis_index("c")
    s = lax.axis_index("s")
    sbufs = [sb0, sb1, sb2, sb3]
    dbufs = [db0, db1, db2, db3]
    rbufs = [r0, r1, r2, r3]
    pltpu.sync_copy(zeros_hbm.at[pl.ds(s * RPS, RPS)],
                    agg_sh.at[pl.ds(s * RPS, RPS)])
    plsc.subcore_barrier()

    ebase = (s * NC + c) * EPW

    def iload(i, b):
        off = ebase + i * CHUNK
        pltpu.async_copy(src_hbm.at[pl.ds(off, CHUNK)], sbufs[b], isem)
        pltpu.async_copy(dst_hbm.at[pl.ds(off, CHUNK)], dbufs[b], isem)

    def iwait(b):
        pltpu.make_async_copy(src_hbm.at[pl.ds(0, CHUNK)], sbufs[b],
                              isem).wait()
        pltpu.make_async_copy(dst_hbm.at[pl.ds(0, CHUNK)], dbufs[b],
                              isem).wait()

    def gstart(b):
        pltpu.async_copy(h_hbm.at[sbufs[b]], rbufs[b], gsem)

    def gwait(b):
        pltpu.make_async_copy(h_hbm.at[sbufs[b]], rbufs[b], gsem).wait()

    for b in range(NBUF):
        iload(b, b)

    def body(g, carry):
        base = g * NBUF
        for b in range(NBUF):
            iwait(b)
            gstart(b)
        for b in range(NBUF):
            gwait(b)
            pltpu.sync_copy(rbufs[b], agg_sh.at[dbufs[b]], add=True)
            nxt = base + NBUF + b

            @pl.when(nxt < NCHUNK)
            def _():
                iload(nxt, b)

        return carry

    lax.fori_loop(0, NFULL, body, 0)
    iwait(0)
    gstart(0)
    gwait(0)
    pltpu.sync_copy(rbufs[0], agg_sh.at[dbufs[0]], add=True)

    plsc.subcore_barrier()
    pltpu.sync_copy(agg_sh.at[pl.ds(s * RPS, RPS)],
                    out_hbm.at[c, pl.ds(s * RPS, RPS)])


def _sc_degcount_body(dst_hbm, zeros_hbm, ones_hbm, out_hbm,
                      db0, db1, db2, db3, ones_v, deg_sh, isem, ssem):
    c = lax.axis_index("c")
    s = lax.axis_index("s")
    dbufs = [db0, db1, db2, db3]
    pltpu.sync_copy(zeros_hbm.at[pl.ds(s * RPS, RPS)],
                    deg_sh.at[pl.ds(s * RPS, RPS)])
    pltpu.sync_copy(ones_hbm, ones_v)
    plsc.subcore_barrier()

    ebase = (s * NC + c) * EPW

    def iload(i, b):
        off = ebase + i * CHUNK
        pltpu.async_copy(dst_hbm.at[pl.ds(off, CHUNK)], dbufs[b], isem)

    def iwait(b):
        pltpu.make_async_copy(dst_hbm.at[pl.ds(0, CHUNK)], dbufs[b],
                              isem).wait()

    def sstart(b):
        return pltpu.async_copy(ones_v, deg_sh.at[dbufs[b]], ssem,
                                add=True)

    for b in range(NBUF):
        iload(b, b)

    def body(g, carry):
        base = g * NBUF
        descs = []
        for b in range(NBUF):
            iwait(b)
            descs.append(sstart(b))
        for b in range(NBUF):
            descs[b].wait()
            nxt = base + NBUF + b

            @pl.when(nxt < NCHUNK)
            def _():
                iload(nxt, b)

        return carry

    lax.fori_loop(0, NFULL, body, 0)
    iwait(0)
    sstart(0).wait()

    plsc.subcore_barrier()
    pltpu.sync_copy(deg_sh.at[pl.ds(s * RPS, RPS)],
                    out_hbm.at[c, pl.ds(s * RPS, RPS)])


@functools.lru_cache(maxsize=None)
def _sc_kernels():
    mesh = plsc.VectorSubcoreMesh(core_axis_name="c", subcore_axis_name="s",
                                  num_cores=NC, num_subcores=NS)
    idx = pltpu.VMEM((CHUNK,), jnp.int32)
    rows = pltpu.VMEM((CHUNK, F), jnp.float32)
    segsum = pl.kernel(
        _sc_segsum_body,
        out_type=jax.ShapeDtypeStruct((NC, NPAD, F), jnp.float32),
        mesh=mesh,
        scratch_types=[idx] * 8 + [rows] * 4 + [
            pltpu.VMEM_SHARED((NPAD, F), jnp.float32),
            pltpu.SemaphoreType.DMA,
            pltpu.SemaphoreType.DMA,
        ],
    )
    degcount = pl.kernel(
        _sc_degcount_body,
        out_type=jax.ShapeDtypeStruct((NC, NPAD, F), jnp.float32),
        mesh=mesh,
        scratch_types=[idx] * 4 + [
            pltpu.VMEM((CHUNK, F), jnp.float32),
            pltpu.VMEM_SHARED((NPAD, F), jnp.float32),
            pltpu.SemaphoreType.DMA,
            pltpu.SemaphoreType.DMA,
        ],
    )
    return segsum, degcount


def _sc_segsum(h, src, dst, zeros):
    return _sc_kernels()[0](h, src, dst, zeros)


def _sc_degcount(dst, zeros):
    ones = jnp.ones((CHUNK, F), jnp.float32)
    return _sc_kernels()[1](dst, zeros, ones)


BLK = 1000
NBLK = N // BLK


def _leaky(v):
    return jnp.where(v >= 0, v, 0.01 * v)


def _tc_layer_body(agg_ref, h_ref, degp_ref, Wl_ref, bl_ref, Wr_ref, out_ref):
    agg = agg_ref[0] + agg_ref[1]
    deg = degp_ref[0, :, 0:1] + degp_ref[1, :, 0:1]
    deg = jnp.clip(deg, 0.0, float(MAXDEG))
    h = h_ref[...]
    acc = jnp.zeros((BLK, F), jnp.float32)
    for d in range(MAXDEG + 1):
        r = (jnp.dot(agg, Wl_ref[d], preferred_element_type=jnp.float32)
             + bl_ref[d][None, :]
             + jnp.dot(h, Wr_ref[d], preferred_element_type=jnp.float32))
        acc = jnp.where(deg == float(d), r, acc)
    out_ref[...] = _leaky(acc)


_tc_layer = pl.pallas_call(
    _tc_layer_body,
    grid=(NBLK,),
    in_specs=[
        pl.BlockSpec((NC, BLK, F), lambda i: (0, i, 0)),
        pl.BlockSpec((BLK, F), lambda i: (i, 0)),
        pl.BlockSpec((NC, BLK, 16), lambda i: (0, i, 0)),
        pl.BlockSpec((MAXDEG + 1, F, F), lambda i: (0, 0, 0)),
        pl.BlockSpec((MAXDEG + 1, F), lambda i: (0, 0)),
        pl.BlockSpec((MAXDEG + 1, F, F), lambda i: (0, 0, 0)),
    ],
    out_specs=pl.BlockSpec((BLK, F), lambda i: (i, 0)),
    out_shape=jax.ShapeDtypeStruct((N, F), jnp.float32),
)


def _tc_pool_body(h_ref, batch_ref, out_ref):
    @pl.when(pl.program_id(0) == 0)
    def _():
        out_ref[...] = jnp.zeros_like(out_ref)

    gids = lax.broadcasted_iota(jnp.int32, (1, NGP), 1)
    oh = (batch_ref[...] == gids).astype(jnp.float32)
    out_ref[...] += lax.dot_general(
        oh, h_ref[...], (((0,), (0,)), ((), ())),
        preferred_element_type=jnp.float32)


_tc_pool = pl.pallas_call(
    _tc_pool_body,
    grid=(NBLK,),
    in_specs=[
        pl.BlockSpec((BLK, F), lambda i: (i, 0)),
        pl.BlockSpec((BLK, 1), lambda i: (i, 0)),
    ],
    out_specs=pl.BlockSpec((NGP, F), lambda i: (0, 0)),
    out_shape=jax.ShapeDtypeStruct((NGP, F), jnp.float32),
)


def _tc_mlp_body(h1_ref, h2_ref, h3_ref, batch_ref, pool_ref,
                 Wc1_ref, bc1_ref, Wcls_ref, bcls_ref, wf_ref, bf_ref,
                 out_ref):
    gids = lax.broadcasted_iota(jnp.int32, (1, NGP), 1)
    oh = (batch_ref[...] == gids).astype(jnp.float32)
    hp = jnp.dot(oh, pool_ref[...], preferred_element_type=jnp.float32)
    cat = jnp.concatenate([h1_ref[...], h2_ref[...], h3_ref[...], hp], axis=1)
    z = (jnp.dot(cat, Wc1_ref[...], preferred_element_type=jnp.float32)
         + bc1_ref[0][None, :])
    for i in range(2):
        z = _leaky(jnp.dot(z, Wcls_ref[i], preferred_element_type=jnp.float32)
                   + bcls_ref[i][None, :])
    o = (jnp.dot(z, wf_ref[...].reshape(F, 1),
                 preferred_element_type=jnp.float32)
         + bf_ref[0, 0])
    o = jax.nn.sigmoid(o)
    out_ref[...] = jnp.broadcast_to(o, (BLK, F))


def _make_mlp():
    return pl.pallas_call(
        _tc_mlp_body,
        grid=(NBLK,),
        in_specs=[
            pl.BlockSpec((BLK, F), lambda i: (i, 0)),
            pl.BlockSpec((BLK, F), lambda i: (i, 0)),
            pl.BlockSpec((BLK, F), lambda i: (i, 0)),
            pl.BlockSpec((BLK, 1), lambda i: (i, 0)),
            pl.BlockSpec((NGP, F), lambda i: (0, 0)),
            pl.BlockSpec((4 * F, F), lambda i: (0, 0)),
            pl.BlockSpec((1, F), lambda i: (0, 0)),
            pl.BlockSpec((2, F, F), lambda i: (0, 0, 0)),
            pl.BlockSpec((2, F), lambda i: (0, 0)),
            pl.BlockSpec((1, F), lambda i: (0, 0)),
            pl.BlockSpec((1, 1), lambda i: (0, 0)),
        ],
        out_specs=pl.BlockSpec((BLK, F), lambda i: (i, 0)),
        out_shape=jax.ShapeDtypeStruct((N, F), jnp.float32),
    )


_tc_mlp = _make_mlp()


def kernel(x, edge_index, edge_attr, batch, conv_Wl, conv_bl, conv_Wr,
           Wc1, bc1, Wcls, bcls, Wf, bf):
    src = edge_index[0]
    dst = edge_index[1]
    zeros_f = jnp.zeros((NPAD, F), jnp.float32)

    degp = _sc_degcount(dst, zeros_f)[:, :, :16]
    batch2 = batch.reshape(N, 1)

    hs = []
    h = x
    for layer in range(3):
        aggp = _sc_segsum(h, src, dst, zeros_f)
        h = _tc_layer(aggp, h, degp, conv_Wl[layer], conv_bl[layer],
                      conv_Wr[layer])
        hs.append(h)

    pool = _tc_pool(hs[2], batch2)

    out = _tc_mlp(hs[0], hs[1], hs[2], batch2, pool, Wc1,
                  bc1.reshape(1, F), Wcls, bcls.reshape(2, F),
                  Wf.reshape(1, F), bf.reshape(1, 1))
    return out[:, :1]
